# baseline (speedup 1.0000x reference)
import functools

import jax
import jax.numpy as jnp
import numpy as np
from jax import lax
from jax.experimental import pallas as pl
from jax.experimental.pallas import tpu as pltpu
from jax.experimental.pallas import tpu_sc as plsc

N = 1000000
C = 50
NBINS = 15

_XQ = np.linspace(0.0, float(N), NBINS + 1).astype(np.float32)
_KS = [int(np.floor(_XQ[i])) for i in range(1, NBINS)]
_FS = np.asarray([float(_XQ[i]) - float(np.floor(_XQ[i]))
                  for i in range(1, NBINS)], dtype=np.float32)
_TARGET_RANKS = np.asarray([0, N - 1] + _KS + [k + 1 for k in _KS], dtype=np.int32)

LO_BITS = 1015021568
ONE_BITS = 1065353216
NBUCKET = 3072
NSLOT = 30
NSUB = 128
SUBW = NSLOT * NSUB

NPAD = 1048576
PER_TILE = 32768
RB = 4000
GRID_A = N // RB


def _softmax_body(lref, labref, cref, aref):
    x = lref[...]
    m = jnp.max(x, axis=1, keepdims=True)
    s = jnp.sum(jnp.exp(x - m), axis=1)
    conf = 1.0 / s
    iota = lax.broadcasted_iota(jnp.int32, (RB, C), 1)
    pred = jnp.min(jnp.where(x == m, iota, C), axis=1)
    cref[...] = conf.reshape(1, 1, RB)
    aref[...] = (pred.reshape(1, 1, RB) == labref[...]).astype(jnp.float32)


def _stage_a(logits, labels3):
    return pl.pallas_call(
        _softmax_body,
        grid=(GRID_A,),
        in_specs=[
            pl.BlockSpec((RB, C), lambda i: (i, 0)),
            pl.BlockSpec((1, 1, RB), lambda i: (i, 0, 0)),
        ],
        out_specs=[
            pl.BlockSpec((1, 1, RB), lambda i: (i, 0, 0)),
            pl.BlockSpec((1, 1, RB), lambda i: (i, 0, 0)),
        ],
        out_shape=[
            jax.ShapeDtypeStruct((GRID_A, 1, RB), jnp.float32),
            jax.ShapeDtypeStruct((GRID_A, 1, RB), jnp.float32),
        ],
    )(logits, labels3)


def _hist1_body(conf_hbm, out_hbm, buf, hist):
    cid = lax.axis_index("c")
    sid = lax.axis_index("s")
    wid = sid * 2 + cid
    lane = lax.iota(jnp.int32, 16)
    lbase = lane * NBUCKET

    def zero(i, _):
        hist[pl.ds(i * 16, 16)] = jnp.zeros((16,), jnp.int32)
        return 0

    lax.fori_loop(0, (NBUCKET * 16) // 16, zero, 0)
    pltpu.sync_copy(conf_hbm.at[pl.ds(wid * PER_TILE, PER_TILE)], buf)
    ones = jnp.ones((16,), jnp.int32)

    def body(v, _):
        bits = buf[pl.ds(v * 16, 16)]
        idx = lax.shift_right_arithmetic(bits - LO_BITS, 14)
        idx = jnp.clip(idx, 0, NBUCKET - 1)
        mask = bits <= ONE_BITS
        plsc.addupdate_scatter(hist, [lbase + idx], ones, mask=mask)
        return 0

    lax.fori_loop(0, PER_TILE // 16, body, 0)
    pltpu.sync_copy(hist, out_hbm.at[wid])


def _stage_b(conf_pad):
    mesh = plsc.VectorSubcoreMesh(core_axis_name="c", subcore_axis_name="s")
    f = pl.kernel(
        _hist1_body,
        out_type=jax.ShapeDtypeStruct((32, 16 * NBUCKET), jnp.int32),
        mesh=mesh,
        compiler_params=pltpu.CompilerParams(needs_layout_passes=False),
        scratch_types=[
            pltpu.VMEM((PER_TILE,), jnp.int32),
            pltpu.VMEM((16 * NBUCKET,), jnp.int32),
        ],
    )
    return f(conf_pad)


def _cum_from_hist(h512):
    Hr = jnp.sum(h512, axis=0)
    k = lax.broadcasted_iota(jnp.int32, (128, 128), 0)
    j = lax.broadcasted_iota(jnp.int32, (128, 128), 1)
    within = jnp.sum(Hr[:, :, None] * (k <= j)[None, :, :].astype(jnp.int32),
                     axis=1)
    rs = jnp.sum(Hr, axis=1)
    r = lax.broadcasted_iota(jnp.int32, (24, 24), 0)
    p = lax.broadcasted_iota(jnp.int32, (24, 24), 1)
    offs = jnp.sum(rs[None, :] * (p < r).astype(jnp.int32), axis=1)
    cum = within + offs[:, None]
    return Hr, cum


def _target_buckets(cum, kt):
    le = (cum[None, :, :] <= kt[:, None, None]).astype(jnp.int32)
    return jnp.sum(le, axis=(1, 2))


def _slot_of(b):
    u = lax.broadcasted_iota(jnp.int32, (NSLOT, NSLOT), 1)
    eq = b[:, None] == b[None, :]
    return jnp.min(jnp.where(eq, u, NSLOT), axis=1)


def _lut_body(href, ktref, out_ref):
    _, cum = _cum_from_hist(href[...])
    kt = ktref[...][0, :NSLOT]
    b = _target_buckets(cum, kt)
    slot = _slot_of(b)
    r2 = lax.broadcasted_iota(jnp.int32, (24, 128), 0)
    c2 = lax.broadcasted_iota(jnp.int32, (24, 128), 1)
    jcell = r2 * 128 + c2
    hit = b[:, None, None] == jcell[None, :, :]
    lut = jnp.min(jnp.where(hit, slot[:, None, None], NSLOT), axis=0)
    lut = jnp.where(lut == NSLOT, -1, lut)
    out_ref[...] = lut.reshape(1, NBUCKET)


def _stage_c1(hist512, kt2d):
    return pl.pallas_call(
        _lut_body,
        out_shape=jax.ShapeDtypeStruct((1, NBUCKET), jnp.int32),
    )(hist512, kt2d)


def _hist2_body(conf_hbm, lut_hbm, out_hbm, buf, lut, sub):
    cid = lax.axis_index("c")
    sid = lax.axis_index("s")
    wid = sid * 2 + cid
    lane = lax.iota(jnp.int32, 16)
    lbase = lane * SUBW

    def zero(i, _):
        sub[pl.ds(i * 16, 16)] = jnp.zeros((16,), jnp.int32)
        return 0

    lax.fori_loop(0, (SUBW * 16) // 16, zero, 0)
    pltpu.sync_copy(lut_hbm, lut)
    pltpu.sync_copy(conf_hbm.at[pl.ds(wid * PER_TILE, PER_TILE)], buf)
    ones = jnp.ones((16,), jnp.int32)

    def body(v, _):
        bits = buf[pl.ds(v * 16, 16)]
        idx = lax.shift_right_arithmetic(bits - LO_BITS, 14)
        idx = jnp.clip(idx, 0, NBUCKET - 1)
        slot = plsc.load_gather(lut, [idx])
        mask = jnp.logical_and(bits <= ONE_BITS, slot >= 0)
        slot = jnp.maximum(slot, 0)
        subidx = jnp.bitwise_and(lax.shift_right_logical(bits, 7), NSUB - 1)
        plsc.addupdate_scatter(sub, [lbase + slot * NSUB + subidx], ones,
                               mask=mask)
        return 0

    lax.fori_loop(0, PER_TILE // 16, body, 0)
    pltpu.sync_copy(sub, out_hbm.at[wid])


def _stage_d(conf_pad, lut_flat):
    mesh = plsc.VectorSubcoreMesh(core_axis_name="c", subcore_axis_name="s")
    f = pl.kernel(
        _hist2_body,
        out_type=jax.ShapeDtypeStruct((32, 16 * SUBW), jnp.int32),
        mesh=mesh,
        compiler_params=pltpu.CompilerParams(needs_layout_passes=False),
        scratch_types=[
            pltpu.VMEM((PER_TILE,), jnp.int32),
            pltpu.VMEM((NBUCKET,), jnp.int32),
            pltpu.VMEM((16 * SUBW,), jnp.int32),
        ],
    )
    return f(conf_pad, lut_flat)


def _bounds_body(href, subref, ktref, fsref, out_ref):
    Hr, cum = _cum_from_hist(href[...])
    kt = ktref[...][0, :NSLOT]
    b = _target_buckets(cum, kt)
    slot = _slot_of(b)
    r2 = lax.broadcasted_iota(jnp.int32, (24, 128), 0)
    c2 = lax.broadcasted_iota(jnp.int32, (24, 128), 1)
    jcell = r2 * 128 + c2
    excl = cum - Hr
    hit = (b[:, None, None] == jcell[None, :, :]).astype(jnp.int32)
    below = jnp.sum(hit * excl[None, :, :], axis=(1, 2))
    rloc = kt - below

    SH = jnp.sum(subref[...], axis=0)
    k = lax.broadcasted_iota(jnp.int32, (NSUB, NSUB), 0)
    j = lax.broadcasted_iota(jnp.int32, (NSUB, NSUB), 1)
    cumsub = jnp.sum(SH[:, :, None] * (k <= j)[None, :, :].astype(jnp.int32),
                     axis=1)
    uu = lax.broadcasted_iota(jnp.int32, (NSLOT, NSLOT), 1)
    sel = (slot[:, None] == uu).astype(jnp.int32)
    cs_t = jnp.sum(sel[:, :, None] * cumsub[None, :, :], axis=1)
    subpos = jnp.sum((cs_t <= rloc[:, None]).astype(jnp.int32), axis=1)

    vbits = (LO_BITS + lax.shift_left(b, 14) + lax.shift_left(subpos, 7) + 64)
    vals = lax.bitcast_convert_type(vbits, jnp.float32)
    low = vals[2:2 + (NBINS - 1)]
    high = vals[2 + (NBINS - 1):2 + 2 * (NBINS - 1)]
    mid = low + fsref[...][0, :NBINS - 1] * (high - low)
    bnd = jnp.concatenate([vals[0:1], mid, vals[1:2],
                           jnp.zeros((112,), jnp.float32)])
    out_ref[...] = bnd.reshape(1, 128)


def _stage_c2(hist512, sub512, kt2d, fs2d):
    return pl.pallas_call(
        _bounds_body,
        out_shape=jax.ShapeDtypeStruct((1, 128), jnp.float32),
    )(hist512, sub512, kt2d, fs2d)


def _ece_body(bref, cref, aref, out_ref, cnt, sc, sa):
    pid = pl.program_id(0)

    @pl.when(pid == 0)
    def _init():
        cnt[...] = jnp.zeros((16, RB), jnp.float32)
        sc[...] = jnp.zeros((16, RB), jnp.float32)
        sa[...] = jnp.zeros((16, RB), jnp.float32)

    c = cref[...].reshape(1, RB)
    a = aref[...].reshape(1, RB)
    for i in range(NBINS):
        lo = bref[0, i]
        hi = bref[0, i + 1]
        m = jnp.logical_and(c > lo, c <= hi).astype(jnp.float32)
        cnt[pl.ds(i, 1), :] += m
        sc[pl.ds(i, 1), :] += m * c
        sa[pl.ds(i, 1), :] += m * a

    @pl.when(pid == GRID_A - 1)
    def _fini():
        n = jnp.sum(cnt[...], axis=1)
        s1 = jnp.sum(sc[...], axis=1)
        s2 = jnp.sum(sa[...], axis=1)
        denom = jnp.maximum(n, 1.0)
        term = jnp.abs(s1 / denom - s2 / denom) * (n / float(N))
        out_ref[0, 0] = jnp.sum(jnp.where(n > 0, term, 0.0))


def _stage_e(bnd, conf3, acc3):
    return pl.pallas_call(
        _ece_body,
        grid=(GRID_A,),
        in_specs=[
            pl.BlockSpec(memory_space=pltpu.SMEM),
            pl.BlockSpec((1, 1, RB), lambda i: (i, 0, 0)),
            pl.BlockSpec((1, 1, RB), lambda i: (i, 0, 0)),
        ],
        out_specs=pl.BlockSpec(memory_space=pltpu.SMEM),
        out_shape=jax.ShapeDtypeStruct((1, 1), jnp.float32),
        scratch_shapes=[
            pltpu.VMEM((16, RB), jnp.float32),
            pltpu.VMEM((16, RB), jnp.float32),
            pltpu.VMEM((16, RB), jnp.float32),
        ],
    )(bnd, conf3, acc3)


@jax.jit
def kernel(logits, labels):
    labels3 = labels.reshape(GRID_A, 1, RB)
    conf3, acc3 = _stage_a(logits, labels3)
    conf_flat = conf3.reshape(N)
    conf_pad = jnp.concatenate(
        [conf_flat, jnp.full((NPAD - N,), 2.0, jnp.float32)])
    conf_bits = lax.bitcast_convert_type(conf_pad, jnp.int32)
    kt2d = jnp.asarray(np.pad(_TARGET_RANKS, (0, 2)).reshape(1, 32))
    fs2d = jnp.asarray(np.pad(_FS, (0, 2)).reshape(1, 16))
    hist = _stage_b(conf_bits).reshape(512, 24, 128)
    lut = _stage_c1(hist, kt2d).reshape(NBUCKET)
    sub = _stage_d(conf_bits, lut).reshape(512, NSLOT, NSUB)
    bnd = _stage_c2(hist, sub, kt2d, fs2d)
    ece = _stage_e(bnd, conf3, acc3)
    return ece.reshape(())
